# speedup vs baseline: 1.8267x; 1.8267x over previous
import functools

import jax
import jax.numpy as jnp
from jax import lax
from jax.experimental import pallas as pl
from jax.experimental.pallas import tpu as pltpu
from jax.experimental.pallas import tpu_sc as plsc

B = 128
V = 16
D = 256
L = 16
NC = 2
NS = 16
NW = NC * NS
BPW = B // NW
NCHUNK = 4
CROWS = (V * V) // NCHUNK
IPC = V // NCHUNK


def _fill_chunk(hv_v, buf, slot, g, write_right):
    for il in range(IPC):
        i = g * IPC + il
        lv = [hv_v[i, pl.ds(c * L, L)] for c in range(D // L)]

        def rbody(r, _):
            row = il * V + r
            for c in range(D // L):
                buf[slot, row, pl.ds(c * L, L)] = lv[c]
            if write_right:
                for c in range(D // L):
                    buf[slot, row, pl.ds(D + c * L, L)] = hv_v[r, pl.ds(c * L, L)]
            return 0

        lax.fori_loop(0, V, rbody, 0, unroll=False)


def _node_to_edge_body(hv_hbm, out_hbm, hv_v, buf, sem_in, sem0, sem1):
    wid = lax.axis_index("s") * NC + lax.axis_index("c")

    def batch_body(bi, _):
        b = wid * BPW + bi
        pltpu.sync_copy(hv_hbm.at[b], hv_v)

        sems = (sem0, sem1)
        copies = [None, None]
        for g in range(NCHUNK):
            slot = g % 2
            if copies[slot] is not None:
                copies[slot].wait()
            _fill_chunk(hv_v, buf, slot, g, write_right=(g < 2))
            copies[slot] = pltpu.async_copy(
                buf.at[slot], out_hbm.at[b, pl.ds(g * CROWS, CROWS)], sems[slot]
            )
        copies[0].wait()
        copies[1].wait()
        return 0

    lax.fori_loop(0, BPW, batch_body, 0, unroll=False)


@jax.jit
def kernel(hv):
    mesh = plsc.VectorSubcoreMesh(core_axis_name="c", subcore_axis_name="s")
    out = pl.kernel(
        _node_to_edge_body,
        out_type=jax.ShapeDtypeStruct((B, V * V, 2 * D), jnp.float32),
        mesh=mesh,
        scratch_types=[
            pltpu.VMEM((V, D), jnp.float32),
            pltpu.VMEM((2, CROWS, 2 * D), jnp.float32),
            pltpu.SemaphoreType.DMA,
            pltpu.SemaphoreType.DMA,
            pltpu.SemaphoreType.DMA,
        ],
    )(hv)
    return out.reshape(B, V, V, 2 * D)
